# Initial kernel scaffold; baseline (speedup 1.0000x reference)
import functools

import jax
import jax.numpy as jnp
from jax import lax
from jax.experimental import pallas as pl
from jax.experimental.pallas import tpu as pltpu
from jax.experimental.pallas import tpu_sc as plsc

_MAXLEN = 1024
_D = 128
_S = 1024
_B = _S * _S
_NC = 2
_NS = 16
_NW = _NC * _NS
_CHUNK = 128
_CPW = _B // (_NW * _CHUNK)
_NBUF = 4
_LANES = 16


def _body(pos_hbm, table_hbm, out_hbm, idx_v, b0, b1, b2, b3, gsem, wsem):
    bufs = [b0, b1, b2, b3]
    wid = lax.axis_index("s") * _NC + lax.axis_index("c")
    row0 = wid * _CPW

    pltpu.sync_copy(pos_hbm.at[pl.ds(row0, _CPW)], idx_v)

    def _fix(i, carry):
        r = i // (_CHUNK // _LANES)
        c = (i % (_CHUNK // _LANES)) * _LANES
        v = idx_v[r, pl.ds(c, _LANES)]
        idx_v[r, pl.ds(c, _LANES)] = (
            jnp.clip(v, -_MAXLEN, _MAXLEN - 1) + _MAXLEN
        )
        return carry

    lax.fori_loop(0, _CPW * (_CHUNK // _LANES), _fix, 0)

    def _group(g, carry):
        j0 = g * _NBUF
        for b in range(_NBUF):
            pltpu.async_copy(table_hbm.at[idx_v.at[j0 + b]], bufs[b], gsem)
        for b in range(_NBUF):
            pltpu.make_async_copy(
                table_hbm.at[idx_v.at[j0 + b]], bufs[b], gsem
            ).wait()
        for b in range(_NBUF):
            dst = out_hbm.at[pl.ds((row0 + j0 + b) * _CHUNK, _CHUNK)]
            pltpu.async_copy(bufs[b], dst, wsem)
        for b in range(_NBUF):
            dst = out_hbm.at[pl.ds((row0 + j0 + b) * _CHUNK, _CHUNK)]
            pltpu.make_async_copy(bufs[b], dst, wsem).wait()
        return carry

    lax.fori_loop(0, _CPW // _NBUF, _group, 0)


_mesh = plsc.VectorSubcoreMesh(core_axis_name="c", subcore_axis_name="s")

_gather = functools.partial(
    pl.kernel,
    out_type=jax.ShapeDtypeStruct((_B, _D), jnp.float32),
    mesh=_mesh,
    scratch_types=[
        pltpu.VMEM((_CPW, _CHUNK), jnp.int32),
        pltpu.VMEM((_CHUNK, _D), jnp.float32),
        pltpu.VMEM((_CHUNK, _D), jnp.float32),
        pltpu.VMEM((_CHUNK, _D), jnp.float32),
        pltpu.VMEM((_CHUNK, _D), jnp.float32),
        pltpu.SemaphoreType.DMA,
        pltpu.SemaphoreType.DMA,
    ],
)(_body)


@jax.jit
def kernel(pos_seq, pe_k):
    pos_flat = pos_seq.reshape(_NW * _CPW, _CHUNK)
    out = _gather(pos_flat, pe_k)
    return out.reshape(_S, _S, _D)


# speedup vs baseline: 5.7474x; 5.7474x over previous
import functools

import jax
import jax.numpy as jnp
from jax import lax
from jax.experimental import pallas as pl
from jax.experimental.pallas import tpu as pltpu
from jax.experimental.pallas import tpu_sc as plsc

_MAXLEN = 1024
_D = 128
_S = 1024
_B = _S * _S
_NC = 2
_NS = 16
_NW = _NC * _NS
_CHUNK = 128
_CPW = _B // (_NW * _CHUNK)
_NBUF = 4
_LANES = 16


def _body(pos_hbm, table_hbm, out_hbm, idx_v, b0, b1, b2, b3, gsem, wsem):
    bufs = [b0, b1, b2, b3]
    wid = lax.axis_index("s") * _NC + lax.axis_index("c")
    row0 = wid * _CPW

    pltpu.sync_copy(pos_hbm.at[pl.ds(row0, _CPW)], idx_v)

    def _fix(i, carry):
        r = i // (_CHUNK // _LANES)
        c = (i % (_CHUNK // _LANES)) * _LANES
        v = idx_v[r, pl.ds(c, _LANES)]
        idx_v[r, pl.ds(c, _LANES)] = (
            jnp.clip(v, -_MAXLEN, _MAXLEN - 1) + _MAXLEN
        )
        return carry

    lax.fori_loop(0, _CPW * (_CHUNK // _LANES), _fix, 0)

    def _group(g, carry):
        j0 = g * _NBUF
        for b in range(_NBUF):
            pltpu.async_copy(table_hbm.at[idx_v.at[j0 + b]], bufs[b], gsem)
        for b in range(_NBUF):
            pltpu.make_async_copy(
                table_hbm.at[idx_v.at[j0 + b]], bufs[b], gsem
            ).wait()
        for b in range(_NBUF):
            dst = out_hbm.at[pl.ds((row0 + j0 + b) * _CHUNK, _CHUNK)]
            pltpu.async_copy(bufs[b], dst, wsem)
        for b in range(_NBUF):
            dst = out_hbm.at[pl.ds((row0 + j0 + b) * _CHUNK, _CHUNK)]
            pltpu.make_async_copy(bufs[b], dst, wsem).wait()
        return carry

    lax.fori_loop(0, _CPW // _NBUF, _group, 0)


@functools.cache
def _build_gather():
    mesh = plsc.VectorSubcoreMesh(
        core_axis_name="c", subcore_axis_name="s",
        num_cores=_NC, num_subcores=_NS,
    )
    return functools.partial(
        pl.kernel,
        out_type=jax.ShapeDtypeStruct((_B, _D), jnp.float32),
        mesh=mesh,
        scratch_types=[
            pltpu.VMEM((_CPW, _CHUNK), jnp.int32),
            pltpu.VMEM((_CHUNK, _D), jnp.float32),
            pltpu.VMEM((_CHUNK, _D), jnp.float32),
            pltpu.VMEM((_CHUNK, _D), jnp.float32),
            pltpu.VMEM((_CHUNK, _D), jnp.float32),
            pltpu.SemaphoreType.DMA,
            pltpu.SemaphoreType.DMA,
        ],
    )(_body)


@jax.jit
def kernel(pos_seq, pe_k):
    pos_flat = pos_seq.reshape(_NW * _CPW, _CHUNK)
    out = _build_gather()(pos_flat, pe_k)
    return out.reshape(_S, _S, _D)
